# Initial kernel scaffold; baseline (speedup 1.0000x reference)
import jax
import jax.numpy as jnp
from jax.experimental import pallas as pl


def kernel(hidden_states, all_pos, seq_start_end, W, b, gamma, beta):
    raise NotImplementedError("write your pallas kernel here")


# speedup vs baseline: 6.4353x; 6.4353x over previous
import functools

import jax
import jax.numpy as jnp
from jax import lax
from jax.experimental import pallas as pl
from jax.experimental.pallas import tpu as pltpu

NS_ = 2.0
GS_ = 8
PH_ = 64
SEG_ = 64
NSEQ_ = 64
BATCH_ = NSEQ_ * SEG_
GRID_ = GS_ * GS_


def _mm_body(h_ref, wr_ref, m_ref):
    m_ref[...] = jnp.dot(h_ref[...], wr_ref[...],
                         preferred_element_type=jnp.float32)


def _stage1_table(h, Wr):
    blk = 512
    return pl.pallas_call(
        _mm_body,
        grid=(BATCH_ // blk,),
        in_specs=[
            pl.BlockSpec((blk, PH_), lambda i: (i, 0)),
            pl.BlockSpec((PH_, GRID_ * PH_), lambda i: (0, 0)),
        ],
        out_specs=pl.BlockSpec((blk, GRID_ * PH_), lambda i: (i, 0)),
        out_shape=jax.ShapeDtypeStruct((BATCH_, GRID_ * PH_), jnp.float32),
    )(h, Wr)


def _pool_body(tab_ref, pos_ref, prx_ref, pry_ref, y_ref):
    xi = pos_ref[:, 0:1]
    yi = pos_ref[:, 1:2]
    prx = prx_ref[0]
    pry = pry_ref[0]
    k = lax.broadcasted_iota(jnp.int32, (SEG_, SEG_ * GRID_), 1)
    c_of_k = k & (GRID_ - 1)
    j_of_k = k >> 6
    i_sub = lax.broadcasted_iota(jnp.int32, (SEG_, SEG_ * GRID_), 0)
    gx = jnp.floor((prx - xi + NS_ / 2) / NS_ * GS_).astype(jnp.int32)
    gy = jnp.floor((yi + NS_ / 2 - pry) / NS_ * GS_).astype(jnp.int32)
    cell = gx + GS_ * gy
    keep = ((cell == c_of_k) & (j_of_k != i_sub)
            & (prx > xi - NS_ / 2) & (prx < xi + NS_ / 2)
            & (pry > yi - NS_ / 2) & (pry < yi + NS_ / 2))
    p = keep.astype(jnp.float32)
    y_ref[...] = jnp.dot(p, tab_ref[...], preferred_element_type=jnp.float32)


def _stage2_pool_tc(table, all_pos, prx, pry):
    return pl.pallas_call(
        _pool_body,
        grid=(NSEQ_,),
        in_specs=[
            pl.BlockSpec((SEG_ * GRID_, PH_), lambda s: (s, 0)),
            pl.BlockSpec((SEG_, 2), lambda s: (s, 0)),
            pl.BlockSpec((1, 1, SEG_ * GRID_), lambda s: (s, 0, 0)),
            pl.BlockSpec((1, 1, SEG_ * GRID_), lambda s: (s, 0, 0)),
        ],
        out_specs=pl.BlockSpec((SEG_, PH_), lambda s: (s, 0)),
        out_shape=jax.ShapeDtypeStruct((BATCH_, PH_), jnp.float32),
    )(table, all_pos, prx, pry)


def _bn_body(y_ref, b_ref, g_ref, be_ref, o_ref):
    y = y_ref[...] + b_ref[...]
    mean = jnp.mean(y, axis=0, keepdims=True)
    var = jnp.mean((y - mean) ** 2, axis=0, keepdims=True)
    yn = (y - mean) * jax.lax.rsqrt(var + 1e-5) * g_ref[...] + be_ref[...]
    o_ref[...] = jnp.maximum(yn, 0.0)


def _stage3_bn(y, b, gamma, beta):
    return pl.pallas_call(
        _bn_body,
        in_specs=[
            pl.BlockSpec((BATCH_, PH_), lambda: (0, 0)),
            pl.BlockSpec((1, PH_), lambda: (0, 0)),
            pl.BlockSpec((1, PH_), lambda: (0, 0)),
            pl.BlockSpec((1, PH_), lambda: (0, 0)),
        ],
        out_specs=pl.BlockSpec((BATCH_, PH_), lambda: (0, 0)),
        out_shape=jax.ShapeDtypeStruct((BATCH_, PH_), jnp.float32),
    )(y, b.reshape(1, PH_), gamma.reshape(1, PH_), beta.reshape(1, PH_))


def kernel(hidden_states, all_pos, seq_start_end, W, b, gamma, beta):
    h = hidden_states.reshape(BATCH_, PH_)
    Wr = W.reshape(PH_, GRID_, PH_).transpose(2, 1, 0).reshape(PH_, GRID_ * PH_)
    table = _stage1_table(h, Wr)
    prx = jnp.repeat(all_pos[:, 0].reshape(NSEQ_, SEG_), GRID_,
                     axis=1).reshape(NSEQ_, 1, SEG_ * GRID_)
    pry = jnp.repeat(all_pos[:, 1].reshape(NSEQ_, SEG_), GRID_,
                     axis=1).reshape(NSEQ_, 1, SEG_ * GRID_)
    y = _stage2_pool_tc(table.reshape(BATCH_ * GRID_, PH_), all_pos, prx, pry)
    return _stage3_bn(y, b, gamma, beta)


# speedup vs baseline: 8.7010x; 1.3521x over previous
import functools

import jax
import jax.numpy as jnp
from jax import lax
from jax.experimental import pallas as pl
from jax.experimental.pallas import tpu as pltpu
from jax.experimental.pallas import tpu_sc as plsc

NS_ = 2.0
GS_ = 8
PH_ = 64
SEG_ = 64
NSEQ_ = 64
BATCH_ = NSEQ_ * SEG_
GRID_ = GS_ * GS_


def _mm_body(h_ref, wr_ref, m_ref):
    m_ref[...] = jnp.dot(h_ref[...], wr_ref[...],
                         preferred_element_type=jnp.float32)


def _stage1_table(h, Wr):
    blk = 512
    return pl.pallas_call(
        _mm_body,
        grid=(BATCH_ // blk,),
        in_specs=[
            pl.BlockSpec((blk, PH_), lambda i: (i, 0)),
            pl.BlockSpec((PH_, GRID_ * PH_), lambda i: (0, 0)),
        ],
        out_specs=pl.BlockSpec((blk, GRID_ * PH_), lambda i: (i, 0)),
        out_shape=jax.ShapeDtypeStruct((BATCH_, GRID_ * PH_), jnp.float32),
    )(h, Wr)


def _pool_body(tab_ref, pos_ref, prx_ref, pry_ref, y_ref):
    xi = pos_ref[:, 0:1]
    yi = pos_ref[:, 1:2]
    prx = prx_ref[0]
    pry = pry_ref[0]
    k = lax.broadcasted_iota(jnp.int32, (SEG_, SEG_ * GRID_), 1)
    c_of_k = k & (GRID_ - 1)
    j_of_k = k >> 6
    i_sub = lax.broadcasted_iota(jnp.int32, (SEG_, SEG_ * GRID_), 0)
    gx = jnp.floor((prx - xi + NS_ / 2) / NS_ * GS_).astype(jnp.int32)
    gy = jnp.floor((yi + NS_ / 2 - pry) / NS_ * GS_).astype(jnp.int32)
    cell = gx + GS_ * gy
    keep = ((cell == c_of_k) & (j_of_k != i_sub)
            & (prx > xi - NS_ / 2) & (prx < xi + NS_ / 2)
            & (pry > yi - NS_ / 2) & (pry < yi + NS_ / 2))
    p = keep.astype(jnp.float32)
    y_ref[...] = jnp.dot(p, tab_ref[...], preferred_element_type=jnp.float32)


def _stage2_pool_tc(table, all_pos, prx, pry):
    return pl.pallas_call(
        _pool_body,
        grid=(NSEQ_,),
        in_specs=[
            pl.BlockSpec((SEG_ * GRID_, PH_), lambda s: (s, 0)),
            pl.BlockSpec((SEG_, 2), lambda s: (s, 0)),
            pl.BlockSpec((1, 1, SEG_ * GRID_), lambda s: (s, 0, 0)),
            pl.BlockSpec((1, 1, SEG_ * GRID_), lambda s: (s, 0, 0)),
        ],
        out_specs=pl.BlockSpec((SEG_, PH_), lambda s: (s, 0)),
        out_shape=jax.ShapeDtypeStruct((BATCH_, PH_), jnp.float32),
    )(table, all_pos, prx, pry)


_NC, _NSC, _L = 2, 16, 16
_NW = _NC * _NSC
_SPW = NSEQ_ // _NW


_DIAG_CELL = GS_ // 2 + GS_ * (GS_ // 2)


def _sc_body(table_hbm, posx_hbm, posy_hbm, y_hbm,
             posx_v, posy_v, idx_v, rows_v, yseg_v, sem0, sem1):
    wid = lax.axis_index("s") * _NC + lax.axis_index("c")

    def compute_idx(seg_base, i, buf):
        xi = posx_v[pl.ds(i, _L)][0]
        yi = posy_v[pl.ds(i, _L)][0]
        diag_row = (seg_base + i) * GRID_ + _DIAG_CELL
        for q in range(SEG_ // _L):
            sl = pl.ds(q * _L, _L)
            xj = posx_v[sl]
            yj = posy_v[sl]
            jloc = lax.iota(jnp.int32, _L) + q * _L
            gx = ((xj - xi + NS_ / 2) * (GS_ / NS_)).astype(jnp.int32)
            gy = ((yi + NS_ / 2 - yj) * (GS_ / NS_)).astype(jnp.int32)
            row = (seg_base + jloc) * GRID_ + gx + GS_ * gy
            row = jnp.minimum(jnp.maximum(row, 0), BATCH_ * GRID_ - 1)
            mask = ((xj <= xi - NS_ / 2) | (xj >= xi + NS_ / 2)
                    | (yj <= yi - NS_ / 2) | (yj >= yi + NS_ / 2)
                    | (jloc == i))
            idx_v[buf, sl] = jnp.where(mask, diag_row, row)

    def fire(buf, sem):
        pltpu.make_async_copy(
            table_hbm.at[idx_v.at[buf]], rows_v.at[buf], sem).start()

    def wait(buf, sem):
        pltpu.make_async_copy(
            table_hbm.at[idx_v.at[buf]], rows_v.at[buf], sem).wait()

    def accum(i, buf):
        def rbody(r, accs):
            return tuple(
                accs[q] + rows_v[buf, r, pl.ds(q * _L, _L)]
                for q in range(PH_ // _L))

        zero = jnp.zeros((_L,), jnp.float32)
        accs = lax.fori_loop(0, SEG_, rbody,
                             (zero,) * (PH_ // _L), unroll=8)
        for q in range(PH_ // _L):
            diag = rows_v[buf, i, pl.ds(q * _L, _L)]
            yseg_v[i, pl.ds(q * _L, _L)] = accs[q] - diag

    for s2 in range(_SPW):
        seg = wid * _SPW + s2
        base = seg * SEG_
        pltpu.sync_copy(posx_hbm.at[pl.ds(base, SEG_)],
                        posx_v.at[pl.ds(0, SEG_)])
        pltpu.sync_copy(posy_hbm.at[pl.ds(base, SEG_)],
                        posy_v.at[pl.ds(0, SEG_)])
        compute_idx(base, 0, 0)
        fire(0, sem0)

        def kbody(k, _, base=base):
            g0 = 2 * k
            compute_idx(base, g0 + 1, 1)
            fire(1, sem1)
            wait(0, sem0)
            accum(g0, 0)
            compute_idx(base, (g0 + 2) & (SEG_ - 1), 0)
            fire(0, sem0)
            wait(1, sem1)
            accum(g0 + 1, 1)
            return 0

        lax.fori_loop(0, SEG_ // 2, kbody, 0)
        wait(0, sem0)
        pltpu.sync_copy(yseg_v, y_hbm.at[pl.ds(base, SEG_), :])


def _stage2_pool_sc(table, posx, posy):
    mesh = plsc.VectorSubcoreMesh(core_axis_name="c", subcore_axis_name="s",
                                  num_cores=_NC, num_subcores=_NSC)
    f = pl.kernel(
        _sc_body,
        out_type=jax.ShapeDtypeStruct((BATCH_, PH_), jnp.float32),
        mesh=mesh,
        compiler_params=pltpu.CompilerParams(use_tc_tiling_on_sc=False),
        scratch_types=[
            pltpu.VMEM((SEG_ + _L,), jnp.float32),
            pltpu.VMEM((SEG_ + _L,), jnp.float32),
            pltpu.VMEM((2, SEG_), jnp.int32),
            pltpu.VMEM((2, SEG_, PH_), jnp.float32),
            pltpu.VMEM((SEG_, PH_), jnp.float32),
            pltpu.SemaphoreType.DMA,
            pltpu.SemaphoreType.DMA,
        ],
    )
    return f(table, posx, posy)


def _bn_body(y_ref, b_ref, g_ref, be_ref, o_ref):
    y = y_ref[...] + b_ref[...]
    mean = jnp.mean(y, axis=0, keepdims=True)
    var = jnp.mean((y - mean) ** 2, axis=0, keepdims=True)
    yn = (y - mean) * jax.lax.rsqrt(var + 1e-5) * g_ref[...] + be_ref[...]
    o_ref[...] = jnp.maximum(yn, 0.0)


def _stage3_bn(y, b, gamma, beta):
    return pl.pallas_call(
        _bn_body,
        in_specs=[
            pl.BlockSpec((BATCH_, PH_), lambda: (0, 0)),
            pl.BlockSpec((1, PH_), lambda: (0, 0)),
            pl.BlockSpec((1, PH_), lambda: (0, 0)),
            pl.BlockSpec((1, PH_), lambda: (0, 0)),
        ],
        out_specs=pl.BlockSpec((BATCH_, PH_), lambda: (0, 0)),
        out_shape=jax.ShapeDtypeStruct((BATCH_, PH_), jnp.float32),
    )(y, b.reshape(1, PH_), gamma.reshape(1, PH_), beta.reshape(1, PH_))


def kernel(hidden_states, all_pos, seq_start_end, W, b, gamma, beta):
    h = hidden_states.reshape(BATCH_, PH_)
    Wr = W.reshape(PH_, GRID_, PH_).transpose(2, 1, 0).reshape(PH_, GRID_ * PH_)
    table = _stage1_table(h, Wr)
    prx = jnp.repeat(all_pos[:, 0].reshape(NSEQ_, SEG_), GRID_,
                     axis=1).reshape(NSEQ_, 1, SEG_ * GRID_)
    pry = jnp.repeat(all_pos[:, 1].reshape(NSEQ_, SEG_), GRID_,
                     axis=1).reshape(NSEQ_, 1, SEG_ * GRID_)
    y = _stage2_pool_sc(table.reshape(BATCH_ * GRID_, PH_),
                        all_pos[:, 0], all_pos[:, 1])
    return _stage3_bn(y, b, gamma, beta)
